# baseline (speedup 1.0000x reference)
import jax
import jax.numpy as jnp
from jax.experimental import pallas as pl
from jax.experimental.pallas import tpu as pltpu

N = 10000
D = 128

B_HEAVY = 80
NBLK = N // B_HEAVY
B_ATT = 2000
NATT = N // B_ATT


def _fused(adja_ref, adja2_ref, x_ref, wa_ref, wa2_ref, wm_ref,
           wk0_ref, wk1_ref, wk2_ref, ava_ref, ava2_ref, avm_ref, av_ref,
           out_ref,
           xa_s, xa2_s, oa_s, oa2_s, cola_s, cola2_s, colm_s, v_s):
    i = pl.program_id(0)

    @pl.when(i == 0)
    def _init():
        wa = wa_ref[...]
        wa2 = wa2_ref[...]

        def body(c, carry):
            rows = pl.ds(c * 1000, 1000)
            xc = x_ref[rows, :]
            xa_s[rows, :] = jnp.dot(xc, wa, precision=jax.lax.Precision.DEFAULT,
                                    preferred_element_type=jnp.float32)
            xa2_s[rows, :] = jnp.dot(xc, wa2, precision=jax.lax.Precision.DEFAULT,
                                     preferred_element_type=jnp.float32)
            return carry

        jax.lax.fori_loop(0, N // 1000, body, 0)
        cola_s[...] = jnp.zeros_like(cola_s)
        cola2_s[...] = jnp.zeros_like(cola2_s)
        colm_s[...] = jnp.zeros_like(colm_s)

    @pl.when(i < NBLK)
    def _heavy():
        rows = pl.ds(i * B_HEAVY, B_HEAVY)

        xm = jnp.dot(x_ref[rows, :], wm_ref[...],
                     precision=jax.lax.Precision.DEFAULT,
                     preferred_element_type=jnp.float32)
        xm = jnp.maximum(xm, 0.0)
        colm_s[...] += jnp.sum(xm, axis=0, keepdims=True)

        oa = jax.lax.dot_general(adja_ref[...], xa_s[...],
                                 (((1,), (0,)), ((), ())),
                                 precision=jax.lax.Precision.DEFAULT,
                                 preferred_element_type=jnp.float32)
        oa = jnp.maximum(oa, 0.0)
        oa_s[rows, :] = oa
        cola_s[...] += jnp.sum(oa, axis=0, keepdims=True)

        oa2 = jax.lax.dot_general(adja2_ref[...], xa2_s[...],
                                  (((1,), (0,)), ((), ())),
                                  precision=jax.lax.Precision.DEFAULT,
                                  preferred_element_type=jnp.float32)
        oa2 = jnp.maximum(oa2, 0.0)
        oa2_s[rows, :] = oa2
        cola2_s[...] += jnp.sum(oa2, axis=0, keepdims=True)

    @pl.when(i == NBLK)
    def _keys():
        inv_n = 1.0 / N

        def v_vec(col_s, wk_ref, att_ref):
            k = jnp.dot(col_s[...] * inv_n, wk_ref[...],
                        preferred_element_type=jnp.float32)
            return jax.lax.dot_general(k, att_ref[...],
                                       (((1,), (1,)), ((), ())),
                                       preferred_element_type=jnp.float32)

        v_s[0:1, :] = v_vec(cola_s, wk0_ref, ava_ref)
        v_s[1:2, :] = v_vec(cola2_s, wk1_ref, ava2_ref)
        v_s[2:3, :] = v_vec(colm_s, wk2_ref, avm_ref)

    @pl.when(i >= NBLK)
    def _attention():
        rows = pl.ds((i - NBLK) * B_ATT, B_ATT)
        v0 = v_s[0:1, :]
        v1 = v_s[1:2, :]
        v2 = v_s[2:3, :]

        oa = oa_s[rows, :]
        oa2 = oa2_s[rows, :]
        xm = jnp.maximum(jnp.dot(x_ref[rows, :], wm_ref[...],
                                 precision=jax.lax.Precision.DEFAULT,
                                 preferred_element_type=jnp.float32), 0.0)

        s0 = jax.nn.sigmoid(jnp.sum(oa * v0, axis=1, keepdims=True))
        s1 = jax.nn.sigmoid(jnp.sum(oa2 * v1, axis=1, keepdims=True))
        s2 = jax.nn.sigmoid(jnp.sum(xm * v2, axis=1, keepdims=True))

        av = av_ref[...]
        z = (s0 * av[0:1, :] + s1 * av[1:2, :] + s2 * av[2:3, :]) * (1.0 / 3.0)
        z = z - jnp.max(z, axis=1, keepdims=True)
        e = jnp.exp(z)
        att = e / jnp.sum(e, axis=1, keepdims=True)

        out_ref[...] = 3.0 * (att[:, 0:1] * oa + att[:, 1:2] * oa2
                              + att[:, 2:3] * xm)


def kernel(inputx, adj_A, adj_A2, weight_A, weight_A2, weight_mlp,
           W_k0, W_k1, W_k2, att_vec_A, att_vec_A2, att_vec_mlp, att_vec):
    f32 = jnp.float32
    last_blk = NBLK - 1

    def adj_map(i):
        return (jnp.minimum(i, last_blk), 0)

    def out_map(i):
        return (jnp.maximum(i - NBLK, 0), 0)

    const = lambda i: (0, 0)

    out = pl.pallas_call(
        _fused,
        grid=(NBLK + NATT,),
        in_specs=[
            pl.BlockSpec((B_HEAVY, N), adj_map),
            pl.BlockSpec((B_HEAVY, N), adj_map),
            pl.BlockSpec((N, D), const),
            pl.BlockSpec((D, D), const),
            pl.BlockSpec((D, D), const),
            pl.BlockSpec((D, D), const),
            pl.BlockSpec((D, D), const),
            pl.BlockSpec((D, D), const),
            pl.BlockSpec((D, D), const),
            pl.BlockSpec((D, D), const),
            pl.BlockSpec((D, D), const),
            pl.BlockSpec((D, D), const),
            pl.BlockSpec((3, 3), const),
        ],
        out_specs=pl.BlockSpec((B_ATT, D), out_map),
        out_shape=jax.ShapeDtypeStruct((N, D), f32),
        scratch_shapes=[
            pltpu.VMEM((N, D), f32),
            pltpu.VMEM((N, D), f32),
            pltpu.VMEM((N, D), f32),
            pltpu.VMEM((N, D), f32),
            pltpu.VMEM((1, D), f32),
            pltpu.VMEM((1, D), f32),
            pltpu.VMEM((1, D), f32),
            pltpu.VMEM((8, D), f32),
        ],
    )(adj_A, adj_A2, inputx, weight_A, weight_A2, weight_mlp,
      W_k0, W_k1, W_k2, att_vec_A, att_vec_A2, att_vec_mlp, att_vec)

    return out


# speedup vs baseline: 1.0630x; 1.0630x over previous
import jax
import jax.numpy as jnp
from jax.experimental import pallas as pl
from jax.experimental.pallas import tpu as pltpu

N = 10000
D = 128

B_HEAVY = 200
NBLK = N // B_HEAVY
B_ATT = 2000
NATT = N // B_ATT


def _fused(adja_ref, adja2_ref, x_ref, wa_ref, wa2_ref, wm_ref,
           wk0_ref, wk1_ref, wk2_ref, ava_ref, ava2_ref, avm_ref, av_ref,
           out_ref,
           xa_s, xa2_s, oa_s, oa2_s, v_s):
    i = pl.program_id(0)

    @pl.when(i == 0)
    def _init():
        wa = wa_ref[...]
        wa2 = wa2_ref[...]

        def body(c, carry):
            rows = pl.ds(c * 1000, 1000)
            xc = x_ref[rows, :]
            xa_s[rows, :] = jnp.dot(xc, wa, precision=jax.lax.Precision.DEFAULT,
                                    preferred_element_type=jnp.float32)
            xa2_s[rows, :] = jnp.dot(xc, wa2, precision=jax.lax.Precision.DEFAULT,
                                     preferred_element_type=jnp.float32)
            return carry

        jax.lax.fori_loop(0, N // 1000, body, 0)

    @pl.when(i < NBLK)
    def _heavy():
        rows = pl.ds(i * B_HEAVY, B_HEAVY)

        oa = jax.lax.dot_general(adja_ref[...], xa_s[...],
                                 (((1,), (0,)), ((), ())),
                                 precision=jax.lax.Precision.DEFAULT,
                                 preferred_element_type=jnp.float32)
        oa_s[rows, :] = jnp.maximum(oa, 0.0).astype(jnp.bfloat16)

        oa2 = jax.lax.dot_general(adja2_ref[...], xa2_s[...],
                                  (((1,), (0,)), ((), ())),
                                  precision=jax.lax.Precision.DEFAULT,
                                  preferred_element_type=jnp.float32)
        oa2_s[rows, :] = jnp.maximum(oa2, 0.0).astype(jnp.bfloat16)

    @pl.when(i == NBLK)
    def _keys():
        inv_n = 1.0 / N
        ones_row = jnp.ones((1, N), jnp.bfloat16)
        cola = jax.lax.dot_general(ones_row, oa_s[...], (((1,), (0,)), ((), ())),
                                   preferred_element_type=jnp.float32)
        cola2 = jax.lax.dot_general(ones_row, oa2_s[...], (((1,), (0,)), ((), ())),
                                    preferred_element_type=jnp.float32)

        def body(c, colm):
            rows = pl.ds(c * 1000, 1000)
            xm = jnp.maximum(jnp.dot(x_ref[rows, :], wm_ref[...],
                                     precision=jax.lax.Precision.DEFAULT,
                                     preferred_element_type=jnp.float32), 0.0)
            return colm + jnp.sum(xm, axis=0, keepdims=True)

        colm = jax.lax.fori_loop(0, N // 1000, body,
                                 jnp.zeros((1, D), jnp.float32))

        def v_vec(col, wk_ref, att_ref):
            k = jnp.dot(col * inv_n, wk_ref[...],
                        preferred_element_type=jnp.float32)
            return jax.lax.dot_general(k, att_ref[...],
                                       (((1,), (1,)), ((), ())),
                                       preferred_element_type=jnp.float32)

        v_s[0:1, :] = v_vec(cola, wk0_ref, ava_ref)
        v_s[1:2, :] = v_vec(cola2, wk1_ref, ava2_ref)
        v_s[2:3, :] = v_vec(colm, wk2_ref, avm_ref)

    @pl.when(i >= NBLK)
    def _attention():
        rows = pl.ds((i - NBLK) * B_ATT, B_ATT)
        v0 = v_s[0:1, :]
        v1 = v_s[1:2, :]
        v2 = v_s[2:3, :]

        oa = oa_s[rows, :].astype(jnp.float32)
        oa2 = oa2_s[rows, :].astype(jnp.float32)
        xm = jnp.maximum(jnp.dot(x_ref[rows, :], wm_ref[...],
                                 precision=jax.lax.Precision.DEFAULT,
                                 preferred_element_type=jnp.float32), 0.0)

        s0 = jax.nn.sigmoid(jnp.sum(oa * v0, axis=1, keepdims=True))
        s1 = jax.nn.sigmoid(jnp.sum(oa2 * v1, axis=1, keepdims=True))
        s2 = jax.nn.sigmoid(jnp.sum(xm * v2, axis=1, keepdims=True))

        av = av_ref[...]
        z = (s0 * av[0:1, :] + s1 * av[1:2, :] + s2 * av[2:3, :]) * (1.0 / 3.0)
        z = z - jnp.max(z, axis=1, keepdims=True)
        e = jnp.exp(z)
        att = e / jnp.sum(e, axis=1, keepdims=True)

        out_ref[...] = 3.0 * (att[:, 0:1] * oa + att[:, 1:2] * oa2
                              + att[:, 2:3] * xm)


def kernel(inputx, adj_A, adj_A2, weight_A, weight_A2, weight_mlp,
           W_k0, W_k1, W_k2, att_vec_A, att_vec_A2, att_vec_mlp, att_vec):
    f32 = jnp.float32
    last_blk = NBLK - 1

    def adj_map(i):
        return (jnp.minimum(i, last_blk), 0)

    def out_map(i):
        return (jnp.maximum(i - NBLK, 0), 0)

    const = lambda i: (0, 0)

    out = pl.pallas_call(
        _fused,
        grid=(NBLK + NATT,),
        in_specs=[
            pl.BlockSpec((B_HEAVY, N), adj_map),
            pl.BlockSpec((B_HEAVY, N), adj_map),
            pl.BlockSpec((N, D), const),
            pl.BlockSpec((D, D), const),
            pl.BlockSpec((D, D), const),
            pl.BlockSpec((D, D), const),
            pl.BlockSpec((D, D), const),
            pl.BlockSpec((D, D), const),
            pl.BlockSpec((D, D), const),
            pl.BlockSpec((D, D), const),
            pl.BlockSpec((D, D), const),
            pl.BlockSpec((D, D), const),
            pl.BlockSpec((3, 3), const),
        ],
        out_specs=pl.BlockSpec((B_ATT, D), out_map),
        out_shape=jax.ShapeDtypeStruct((N, D), f32),
        scratch_shapes=[
            pltpu.VMEM((N, D), f32),
            pltpu.VMEM((N, D), f32),
            pltpu.VMEM((N, D), jnp.bfloat16),
            pltpu.VMEM((N, D), jnp.bfloat16),
            pltpu.VMEM((8, D), f32),
        ],
    )(adj_A, adj_A2, inputx, weight_A, weight_A2, weight_mlp,
      W_k0, W_k1, W_k2, att_vec_A, att_vec_A2, att_vec_mlp, att_vec)

    return out


# speedup vs baseline: 1.0811x; 1.0170x over previous
import jax
import jax.numpy as jnp
from jax.experimental import pallas as pl
from jax.experimental.pallas import tpu as pltpu

N = 10000
D = 128

B_HEAVY = 200
NBLK = N // B_HEAVY
B_ATT = 2000
NATT = N // B_ATT


def _fused(adja_ref, adja2_ref, x_ref, wa_ref, wa2_ref, wm_ref,
           wk0_ref, wk1_ref, wk2_ref, ava_ref, ava2_ref, avm_ref, av_ref,
           out_ref,
           xa_s, xa2_s, oa_s, oa2_s, cola_s, cola2_s, colm_s, v_s):
    i = pl.program_id(0)

    @pl.when(i == 0)
    def _init():
        wa = wa_ref[...]
        wa2 = wa2_ref[...]

        def body(c, carry):
            rows = pl.ds(c * 1000, 1000)
            xc = x_ref[rows, :]
            xa_s[rows, :] = jnp.dot(xc, wa, precision=jax.lax.Precision.DEFAULT,
                                    preferred_element_type=jnp.float32)
            xa2_s[rows, :] = jnp.dot(xc, wa2, precision=jax.lax.Precision.DEFAULT,
                                     preferred_element_type=jnp.float32)
            return carry

        jax.lax.fori_loop(0, N // 1000, body, 0)
        cola_s[...] = jnp.zeros_like(cola_s)
        cola2_s[...] = jnp.zeros_like(cola2_s)
        colm_s[...] = jnp.zeros_like(colm_s)

    @pl.when(i < NBLK)
    def _heavy():
        rows = pl.ds(i * B_HEAVY, B_HEAVY)

        xm = jnp.dot(x_ref[rows, :], wm_ref[...],
                     precision=jax.lax.Precision.DEFAULT,
                     preferred_element_type=jnp.float32)
        xm = jnp.maximum(xm, 0.0)
        colm_s[...] += jnp.sum(xm, axis=0, keepdims=True)

        oa = jax.lax.dot_general(adja_ref[...], xa_s[...],
                                 (((1,), (0,)), ((), ())),
                                 precision=jax.lax.Precision.DEFAULT,
                                 preferred_element_type=jnp.float32)
        oa = jnp.maximum(oa, 0.0)
        oa_s[rows, :] = oa.astype(jnp.bfloat16)
        cola_s[...] += jnp.sum(oa, axis=0, keepdims=True)

        oa2 = jax.lax.dot_general(adja2_ref[...], xa2_s[...],
                                  (((1,), (0,)), ((), ())),
                                  precision=jax.lax.Precision.DEFAULT,
                                  preferred_element_type=jnp.float32)
        oa2 = jnp.maximum(oa2, 0.0)
        oa2_s[rows, :] = oa2.astype(jnp.bfloat16)
        cola2_s[...] += jnp.sum(oa2, axis=0, keepdims=True)

    @pl.when(i == NBLK)
    def _keys():
        inv_n = 1.0 / N

        def v_vec(col_s, wk_ref, att_ref):
            k = jnp.dot(col_s[...] * inv_n, wk_ref[...],
                        preferred_element_type=jnp.float32)
            return jax.lax.dot_general(k, att_ref[...],
                                       (((1,), (1,)), ((), ())),
                                       preferred_element_type=jnp.float32)

        v_s[0:1, :] = v_vec(cola_s, wk0_ref, ava_ref)
        v_s[1:2, :] = v_vec(cola2_s, wk1_ref, ava2_ref)
        v_s[2:3, :] = v_vec(colm_s, wk2_ref, avm_ref)

    @pl.when(i >= NBLK)
    def _attention():
        rows = pl.ds((i - NBLK) * B_ATT, B_ATT)
        v0 = v_s[0:1, :]
        v1 = v_s[1:2, :]
        v2 = v_s[2:3, :]

        oa = oa_s[rows, :].astype(jnp.float32)
        oa2 = oa2_s[rows, :].astype(jnp.float32)
        xm = jnp.maximum(jnp.dot(x_ref[rows, :], wm_ref[...],
                                 precision=jax.lax.Precision.DEFAULT,
                                 preferred_element_type=jnp.float32), 0.0)

        s0 = jax.nn.sigmoid(jnp.sum(oa * v0, axis=1, keepdims=True))
        s1 = jax.nn.sigmoid(jnp.sum(oa2 * v1, axis=1, keepdims=True))
        s2 = jax.nn.sigmoid(jnp.sum(xm * v2, axis=1, keepdims=True))

        av = av_ref[...]
        z = (s0 * av[0:1, :] + s1 * av[1:2, :] + s2 * av[2:3, :]) * (1.0 / 3.0)
        z = z - jnp.max(z, axis=1, keepdims=True)
        e = jnp.exp(z)
        att = e / jnp.sum(e, axis=1, keepdims=True)

        out_ref[...] = 3.0 * (att[:, 0:1] * oa + att[:, 1:2] * oa2
                              + att[:, 2:3] * xm)


def kernel(inputx, adj_A, adj_A2, weight_A, weight_A2, weight_mlp,
           W_k0, W_k1, W_k2, att_vec_A, att_vec_A2, att_vec_mlp, att_vec):
    f32 = jnp.float32
    last_blk = NBLK - 1

    def adj_map(i):
        return (jnp.minimum(i, last_blk), 0)

    def out_map(i):
        return (jnp.maximum(i - NBLK, 0), 0)

    const = lambda i: (0, 0)

    out = pl.pallas_call(
        _fused,
        grid=(NBLK + NATT,),
        in_specs=[
            pl.BlockSpec((B_HEAVY, N), adj_map),
            pl.BlockSpec((B_HEAVY, N), adj_map),
            pl.BlockSpec((N, D), const),
            pl.BlockSpec((D, D), const),
            pl.BlockSpec((D, D), const),
            pl.BlockSpec((D, D), const),
            pl.BlockSpec((D, D), const),
            pl.BlockSpec((D, D), const),
            pl.BlockSpec((D, D), const),
            pl.BlockSpec((D, D), const),
            pl.BlockSpec((D, D), const),
            pl.BlockSpec((D, D), const),
            pl.BlockSpec((3, 3), const),
        ],
        out_specs=pl.BlockSpec((B_ATT, D), out_map),
        out_shape=jax.ShapeDtypeStruct((N, D), f32),
        scratch_shapes=[
            pltpu.VMEM((N, D), f32),
            pltpu.VMEM((N, D), f32),
            pltpu.VMEM((N, D), jnp.bfloat16),
            pltpu.VMEM((N, D), jnp.bfloat16),
            pltpu.VMEM((1, D), f32),
            pltpu.VMEM((1, D), f32),
            pltpu.VMEM((1, D), f32),
            pltpu.VMEM((8, D), f32),
        ],
    )(adj_A, adj_A2, inputx, weight_A, weight_A2, weight_mlp,
      W_k0, W_k1, W_k2, att_vec_A, att_vec_A2, att_vec_mlp, att_vec)

    return out
